# baseline (speedup 1.0000x reference)
import jax
import jax.numpy as jnp
from jax.experimental import pallas as pl
from jax.experimental.pallas import tpu as pltpu

Q = 256


def _silu(v):
    return v * jax.nn.sigmoid(v)


def kernel(x, W_in, conv_w, conv_b, dt_bias, A_log, D, norm_w, W_out):
    Bsz, L, d_model = x.shape
    d_in_proj = W_in.shape[1]
    conv_dim, d_conv = conv_w.shape
    H = A_log.shape[0]
    d_inner = norm_w.shape[0]
    P = d_inner // H
    N = (conv_dim - d_inner) // 2
    d_out = W_out.shape[1]
    LT = L // Q
    M_total = Bsz * L

    xf = x.reshape(M_total, d_model)
    W_in_bf = W_in.astype(jnp.bfloat16)
    W_out_bf = W_out.astype(jnp.bfloat16)
    cwT = conv_w.T
    cb2 = conv_b.reshape(1, conv_dim)
    dtb2 = dt_bias.reshape(1, H)
    Av = -jnp.exp(A_log)
    a_row2 = Av.reshape(1, H)
    e64 = jnp.kron(jnp.eye(H, dtype=jnp.float32),
                   jnp.ones((1, P), jnp.float32)).astype(jnp.bfloat16)
    drep = jnp.repeat(D, P).reshape(1, d_inner)
    nw2 = norm_w.reshape(1, d_inner)

    def body(x_ref, w_ref, cw_ref, cb_ref, dtb_ref, ar_ref, e64_ref, dv_ref,
             nw_ref, wo_ref, o_ref, xtail, state, xsc, zbuf, ybuf):
        c = pl.program_id(1)

        zxb = jnp.dot(x_ref[...].astype(jnp.bfloat16), w_ref[...],
                      preferred_element_type=jnp.float32)
        z = zxb[:, :d_inner]
        zbuf[...] = _silu(z)
        dtp = zxb[:, d_inner + conv_dim:] + dtb_ref[...]
        dtc = jnp.maximum(dtp, 0.0) + jnp.log(1.0 + jnp.exp(-jnp.abs(dtp)))
        raw = zxb[:, d_inner:d_inner + conv_dim]

        @pl.when(c == 0)
        def _():
            xtail[...] = jnp.zeros_like(xtail)
            state[...] = jnp.zeros_like(state)

        tail = xtail[...]
        xc = raw * cw_ref[d_conv - 1:d_conv, :]
        for k in range(d_conv - 1):
            sh = d_conv - 1 - k
            shifted = jnp.concatenate([tail[8 - sh:, :], raw[:Q - sh, :]], axis=0)
            xc = xc + shifted * cw_ref[k:k + 1, :]
        xsc[...] = _silu(xc + cb_ref[...])
        xtail[...] = raw[Q - 8:, :]

        q = 64
        SC = Q // q
        a_colH = dtc * ar_ref[...]
        ir = jax.lax.broadcasted_iota(jnp.int32, (q, q), 0)
        ic = jax.lax.broadcasted_iota(jnp.int32, (q, q), 1)
        causal = ir >= ic
        Tmat = causal.astype(jnp.float32)
        Umat = (ir <= ic).astype(jnp.float32)
        e64v = e64_ref[...]
        G = 256 // P

        for s in range(SC):
            rs = slice(s * q, (s + 1) * q)
            Bs = xsc[rs, d_inner:d_inner + N]
            Cs = xsc[rs, d_inner + N:]
            CBs = jax.lax.dot_general(Cs, Bs, (((1,), (1,)), ((), ())),
                                      preferred_element_type=jnp.float32)
            CBm = jnp.where(causal, CBs, 0.0)
            a_s = a_colH[rs]
            dt_s = dtc[rs]
            CumCol = jnp.dot(Tmat, a_s, preferred_element_type=jnp.float32)
            CumRow = jax.lax.dot_general(a_s, Umat, (((0,), (0,)), ((), ())),
                                         preferred_element_type=jnp.float32)
            EcolAll = jnp.exp(CumCol)
            DeclastAll = jnp.exp(CumCol[q - 1:q, :] - CumCol)
            dtrep = jnp.dot(dt_s.astype(jnp.bfloat16), e64v,
                            preferred_element_type=jnp.float32)
            v2rep = jnp.dot((dt_s * DeclastAll).astype(jnp.bfloat16), e64v,
                            preferred_element_type=jnp.float32)
            ecolrep = jnp.dot(EcolAll.astype(jnp.bfloat16), e64v,
                              preferred_element_type=jnp.float32)

            for g in range(H // G):
                gs = slice(g * G * P, (g + 1) * G * P)
                xg = xsc[rs, gs]
                Xw4 = xg * dtrep[:, gs]
                S4 = state[g]
                yinter4 = ecolrep[:, gs] * jnp.dot(Cs, S4,
                                                   preferred_element_type=jnp.float32)
                XwD4 = xg * v2rep[:, gs]
                Snew4 = jax.lax.dot_general(Bs, XwD4, (((0,), (0,)), ((), ())),
                                            preferred_element_type=jnp.float32)
                state[g] = Snew4 + ecolrep[q - 1:q, gs] * S4
                for j in range(G):
                    h = g * G + j
                    hs = slice(h * P, (h + 1) * P)
                    js = slice(j * P, (j + 1) * P)
                    diff = CumCol[:, h:h + 1] - CumRow[h:h + 1, :]
                    E = jnp.exp(jnp.minimum(diff, 0.0))
                    M = CBm * E
                    ybuf[rs, hs] = (jnp.dot(M, Xw4[:, js],
                                            preferred_element_type=jnp.float32)
                                    + yinter4[:, js])

        u = (ybuf[...] + xsc[:, :d_inner] * dv_ref[...]) * zbuf[...]
        ms = jnp.mean(u * u, axis=1, keepdims=True)
        u = u * jax.lax.rsqrt(ms + 1e-5) * nw_ref[...]
        o_ref[...] = jnp.dot(u.astype(jnp.bfloat16), wo_ref[...],
                             preferred_element_type=jnp.float32)

    out = pl.pallas_call(
        body,
        grid=(Bsz, LT),
        in_specs=[
            pl.BlockSpec((Q, d_model), lambda b, c: (b * LT + c, 0)),
            pl.BlockSpec((d_model, d_in_proj), lambda b, c: (0, 0)),
            pl.BlockSpec((d_conv, conv_dim), lambda b, c: (0, 0)),
            pl.BlockSpec((1, conv_dim), lambda b, c: (0, 0)),
            pl.BlockSpec((1, H), lambda b, c: (0, 0)),
            pl.BlockSpec((1, H), lambda b, c: (0, 0)),
            pl.BlockSpec((H, d_inner), lambda b, c: (0, 0)),
            pl.BlockSpec((1, d_inner), lambda b, c: (0, 0)),
            pl.BlockSpec((1, d_inner), lambda b, c: (0, 0)),
            pl.BlockSpec((d_inner, d_out), lambda b, c: (0, 0)),
        ],
        out_specs=pl.BlockSpec((Q, d_out), lambda b, c: (b * LT + c, 0)),
        out_shape=jax.ShapeDtypeStruct((M_total, d_out), jnp.float32),
        scratch_shapes=[
            pltpu.VMEM((8, conv_dim), jnp.float32),
            pltpu.VMEM((H * P // 256, N, 256), jnp.float32),
            pltpu.VMEM((Q, conv_dim), jnp.float32),
            pltpu.VMEM((Q, d_inner), jnp.float32),
            pltpu.VMEM((Q, d_inner), jnp.float32),
        ],
        compiler_params=pltpu.CompilerParams(
            dimension_semantics=("parallel", "arbitrary"),
            vmem_limit_bytes=56 * 1024 * 1024,
            fuse_transposed_lhs_in_matmul=True,
            flags={"XLA_TPU_STORE_TO_LOAD_FORWARDING_WINDOW": 12288},
        ),
        name="mamba2_fused",
    )(xf, W_in_bf, cwT, cb2, dtb2, a_row2, e64, drep, nw2, W_out_bf)

    return out.reshape(Bsz, L, d_out)


# speedup vs baseline: 1.0001x; 1.0001x over previous
import jax
import jax.numpy as jnp
from jax.experimental import pallas as pl
from jax.experimental.pallas import tpu as pltpu

Q = 256


def _silu(v):
    return v * jax.nn.sigmoid(v)


def kernel(x, W_in, conv_w, conv_b, dt_bias, A_log, D, norm_w, W_out):
    Bsz, L, d_model = x.shape
    d_in_proj = W_in.shape[1]
    conv_dim, d_conv = conv_w.shape
    H = A_log.shape[0]
    d_inner = norm_w.shape[0]
    P = d_inner // H
    N = (conv_dim - d_inner) // 2
    d_out = W_out.shape[1]
    LT = L // Q
    M_total = Bsz * L

    xf = x.reshape(M_total, d_model)
    W_in_bf = W_in.astype(jnp.bfloat16)
    W_out_bf = W_out.astype(jnp.bfloat16)
    cwT = conv_w.T
    cb2 = conv_b.reshape(1, conv_dim)
    dtb2 = dt_bias.reshape(1, H)
    Av = -jnp.exp(A_log)
    a_row2 = Av.reshape(1, H)
    e64 = jnp.kron(jnp.eye(H, dtype=jnp.float32),
                   jnp.ones((1, P), jnp.float32)).astype(jnp.bfloat16)
    drep = jnp.repeat(D, P).reshape(1, d_inner)
    nw2 = norm_w.reshape(1, d_inner)

    def body(x_ref, w_ref, cw_ref, cb_ref, dtb_ref, ar_ref, e64_ref, dv_ref,
             nw_ref, wo_ref, o_ref, xtail, state, xsc, zbuf, ybuf):
        c = pl.program_id(1)

        zxb = jnp.dot(x_ref[...].astype(jnp.bfloat16), w_ref[...],
                      preferred_element_type=jnp.float32)
        z = zxb[:, :d_inner]
        zbuf[...] = _silu(z)
        dtp = zxb[:, d_inner + conv_dim:] + dtb_ref[...]
        dtc = jnp.maximum(dtp, 0.0) + jnp.log(1.0 + jnp.exp(-jnp.abs(dtp)))
        raw = zxb[:, d_inner:d_inner + conv_dim]

        @pl.when(c == 0)
        def _():
            xtail[...] = jnp.zeros_like(xtail)
            state[...] = jnp.zeros_like(state)

        tail = xtail[...]
        xc = raw * cw_ref[d_conv - 1:d_conv, :]
        for k in range(d_conv - 1):
            sh = d_conv - 1 - k
            shifted = jnp.concatenate([tail[8 - sh:, :], raw[:Q - sh, :]], axis=0)
            xc = xc + shifted * cw_ref[k:k + 1, :]
        xsc[...] = _silu(xc + cb_ref[...])
        xtail[...] = raw[Q - 8:, :]

        q = 64
        SC = Q // q
        a_colH = dtc * ar_ref[...]
        ir = jax.lax.broadcasted_iota(jnp.int32, (q, q), 0)
        ic = jax.lax.broadcasted_iota(jnp.int32, (q, q), 1)
        causal = ir >= ic
        Tmat = causal.astype(jnp.float32)
        Umat = (ir <= ic).astype(jnp.float32)
        e64v = e64_ref[...]
        G = 256 // P

        for s in range(SC):
            rs = slice(s * q, (s + 1) * q)
            Bs = xsc[rs, d_inner:d_inner + N]
            Cs = xsc[rs, d_inner + N:]
            CBs = jax.lax.dot_general(Cs, Bs, (((1,), (1,)), ((), ())),
                                      preferred_element_type=jnp.float32)
            CBm = jnp.where(causal, CBs, 0.0)
            a_s = a_colH[rs]
            dt_s = dtc[rs]
            CumCol = jnp.dot(Tmat, a_s, preferred_element_type=jnp.float32)
            CumRow = jax.lax.dot_general(a_s, Umat, (((0,), (0,)), ((), ())),
                                         preferred_element_type=jnp.float32)
            EcolAll = jnp.exp(CumCol)
            DeclastAll = jnp.exp(CumCol[q - 1:q, :] - CumCol)
            dtrep = jnp.dot(dt_s.astype(jnp.bfloat16), e64v,
                            preferred_element_type=jnp.float32)
            v2rep = jnp.dot((dt_s * DeclastAll).astype(jnp.bfloat16), e64v,
                            preferred_element_type=jnp.float32)
            ecolrep = jnp.dot(EcolAll.astype(jnp.bfloat16), e64v,
                              preferred_element_type=jnp.float32)

            for g in range(H // G):
                gs = slice(g * G * P, (g + 1) * G * P)
                xg = xsc[rs, gs]
                Xw4 = xg * dtrep[:, gs]
                S4 = state[g]
                yinter4 = ecolrep[:, gs] * jnp.dot(Cs, S4,
                                                   preferred_element_type=jnp.float32)
                XwD4 = xg * v2rep[:, gs]
                Snew4 = jax.lax.dot_general(Bs, XwD4, (((0,), (0,)), ((), ())),
                                            preferred_element_type=jnp.float32)
                state[g] = Snew4 + ecolrep[q - 1:q, gs] * S4
                for j in range(G):
                    h = g * G + j
                    hs = slice(h * P, (h + 1) * P)
                    js = slice(j * P, (j + 1) * P)
                    diff = CumCol[:, h:h + 1] - CumRow[h:h + 1, :]
                    E = jnp.exp(jnp.minimum(diff, 0.0))
                    M = CBm * E
                    ybuf[rs, hs] = (jnp.dot(M, Xw4[:, js],
                                            preferred_element_type=jnp.float32)
                                    + yinter4[:, js])

        u = (ybuf[...] + xsc[:, :d_inner] * dv_ref[...]) * zbuf[...]
        ms = jnp.mean(u * u, axis=1, keepdims=True)
        u = u * jax.lax.rsqrt(ms + 1e-5) * nw_ref[...]
        o_ref[...] = jnp.dot(u.astype(jnp.bfloat16), wo_ref[...],
                             preferred_element_type=jnp.float32)

    out = pl.pallas_call(
        body,
        grid=(Bsz, LT),
        in_specs=[
            pl.BlockSpec((Q, d_model), lambda b, c: (b * LT + c, 0)),
            pl.BlockSpec((d_model, d_in_proj), lambda b, c: (0, 0)),
            pl.BlockSpec((d_conv, conv_dim), lambda b, c: (0, 0)),
            pl.BlockSpec((1, conv_dim), lambda b, c: (0, 0)),
            pl.BlockSpec((1, H), lambda b, c: (0, 0)),
            pl.BlockSpec((1, H), lambda b, c: (0, 0)),
            pl.BlockSpec((H, d_inner), lambda b, c: (0, 0)),
            pl.BlockSpec((1, d_inner), lambda b, c: (0, 0)),
            pl.BlockSpec((1, d_inner), lambda b, c: (0, 0)),
            pl.BlockSpec((d_inner, d_out), lambda b, c: (0, 0)),
        ],
        out_specs=pl.BlockSpec((Q, d_out), lambda b, c: (b * LT + c, 0)),
        out_shape=jax.ShapeDtypeStruct((M_total, d_out), jnp.float32),
        scratch_shapes=[
            pltpu.VMEM((8, conv_dim), jnp.float32),
            pltpu.VMEM((H * P // 256, N, 256), jnp.float32),
            pltpu.VMEM((Q, conv_dim), jnp.float32),
            pltpu.VMEM((Q, d_inner), jnp.float32),
            pltpu.VMEM((Q, d_inner), jnp.float32),
        ],
        compiler_params=pltpu.CompilerParams(
            dimension_semantics=("parallel", "arbitrary"),
            vmem_limit_bytes=56 * 1024 * 1024,
            fuse_transposed_lhs_in_matmul=True,
        ),
        name="mamba2_fused",
    )(xf, W_in_bf, cwT, cb2, dtb2, a_row2, e64, drep, nw2, W_out_bf)

    return out.reshape(Bsz, L, d_out)


# speedup vs baseline: 1.0205x; 1.0204x over previous
import jax
import jax.numpy as jnp
from jax.experimental import pallas as pl
from jax.experimental.pallas import tpu as pltpu

Q = 512


def _silu(v):
    return v * jax.nn.sigmoid(v)


def kernel(x, W_in, conv_w, conv_b, dt_bias, A_log, D, norm_w, W_out):
    Bsz, L, d_model = x.shape
    d_in_proj = W_in.shape[1]
    conv_dim, d_conv = conv_w.shape
    H = A_log.shape[0]
    d_inner = norm_w.shape[0]
    P = d_inner // H
    N = (conv_dim - d_inner) // 2
    d_out = W_out.shape[1]
    LT = L // Q
    M_total = Bsz * L

    xf = x.reshape(M_total, d_model)
    W_in_bf = W_in.astype(jnp.bfloat16)
    W_out_bf = W_out.astype(jnp.bfloat16)
    cwT = conv_w.T
    cb2 = conv_b.reshape(1, conv_dim)
    dtb2 = dt_bias.reshape(1, H)
    Av = -jnp.exp(A_log)
    a_row2 = Av.reshape(1, H)
    e64 = jnp.kron(jnp.eye(H, dtype=jnp.float32),
                   jnp.ones((1, P), jnp.float32)).astype(jnp.bfloat16)
    drep = jnp.repeat(D, P).reshape(1, d_inner)
    nw2 = norm_w.reshape(1, d_inner)

    def body(x_ref, w_ref, cw_ref, cb_ref, dtb_ref, ar_ref, e64_ref, dv_ref,
             nw_ref, wo_ref, o_ref, xtail, state, xsc, zbuf, ybuf):
        c = pl.program_id(1)

        zxb = jnp.dot(x_ref[...].astype(jnp.bfloat16), w_ref[...],
                      preferred_element_type=jnp.float32)
        z = zxb[:, :d_inner]
        zbuf[...] = _silu(z)
        dtp = zxb[:, d_inner + conv_dim:] + dtb_ref[...]
        dtc = jnp.maximum(dtp, 0.0) + jnp.log(1.0 + jnp.exp(-jnp.abs(dtp)))
        raw = zxb[:, d_inner:d_inner + conv_dim]

        @pl.when(c == 0)
        def _():
            xtail[...] = jnp.zeros_like(xtail)
            state[...] = jnp.zeros_like(state)

        tail = xtail[...]
        xc = raw * cw_ref[d_conv - 1:d_conv, :]
        for k in range(d_conv - 1):
            sh = d_conv - 1 - k
            shifted = jnp.concatenate([tail[8 - sh:, :], raw[:Q - sh, :]], axis=0)
            xc = xc + shifted * cw_ref[k:k + 1, :]
        xsc[...] = _silu(xc + cb_ref[...])
        xtail[...] = raw[Q - 8:, :]

        q = 64
        SC = Q // q
        a_colH = dtc * ar_ref[...]
        ir = jax.lax.broadcasted_iota(jnp.int32, (q, q), 0)
        ic = jax.lax.broadcasted_iota(jnp.int32, (q, q), 1)
        causal = ir >= ic
        Tmat = causal.astype(jnp.float32)
        Umat = (ir <= ic).astype(jnp.float32)
        e64v = e64_ref[...]
        G = 256 // P

        for s in range(SC):
            rs = slice(s * q, (s + 1) * q)
            Bs = xsc[rs, d_inner:d_inner + N]
            Cs = xsc[rs, d_inner + N:]
            CBs = jax.lax.dot_general(Cs, Bs, (((1,), (1,)), ((), ())),
                                      preferred_element_type=jnp.float32)
            CBm = jnp.where(causal, CBs, 0.0)
            a_s = a_colH[rs]
            dt_s = dtc[rs]
            CumCol = jnp.dot(Tmat, a_s, preferred_element_type=jnp.float32)
            CumRow = jax.lax.dot_general(a_s, Umat, (((0,), (0,)), ((), ())),
                                         preferred_element_type=jnp.float32)
            EcolAll = jnp.exp(CumCol)
            DeclastAll = jnp.exp(CumCol[q - 1:q, :] - CumCol)
            dtrep = jnp.dot(dt_s.astype(jnp.bfloat16), e64v,
                            preferred_element_type=jnp.float32)
            v2rep = jnp.dot((dt_s * DeclastAll).astype(jnp.bfloat16), e64v,
                            preferred_element_type=jnp.float32)
            ecolrep = jnp.dot(EcolAll.astype(jnp.bfloat16), e64v,
                              preferred_element_type=jnp.float32)

            for g in range(H // G):
                gs = slice(g * G * P, (g + 1) * G * P)
                xg = xsc[rs, gs]
                Xw4 = xg * dtrep[:, gs]
                S4 = state[g]
                yinter4 = ecolrep[:, gs] * jnp.dot(Cs, S4,
                                                   preferred_element_type=jnp.float32)
                XwD4 = xg * v2rep[:, gs]
                Snew4 = jax.lax.dot_general(Bs, XwD4, (((0,), (0,)), ((), ())),
                                            preferred_element_type=jnp.float32)
                state[g] = Snew4 + ecolrep[q - 1:q, gs] * S4
                for j in range(G):
                    h = g * G + j
                    hs = slice(h * P, (h + 1) * P)
                    js = slice(j * P, (j + 1) * P)
                    diff = CumCol[:, h:h + 1] - CumRow[h:h + 1, :]
                    E = jnp.exp(jnp.minimum(diff, 0.0))
                    M = CBm * E
                    ybuf[rs, hs] = (jnp.dot(M, Xw4[:, js],
                                            preferred_element_type=jnp.float32)
                                    + yinter4[:, js])

        u = (ybuf[...] + xsc[:, :d_inner] * dv_ref[...]) * zbuf[...]
        ms = jnp.mean(u * u, axis=1, keepdims=True)
        u = u * jax.lax.rsqrt(ms + 1e-5) * nw_ref[...]
        o_ref[...] = jnp.dot(u.astype(jnp.bfloat16), wo_ref[...],
                             preferred_element_type=jnp.float32)

    out = pl.pallas_call(
        body,
        grid=(Bsz, LT),
        in_specs=[
            pl.BlockSpec((Q, d_model), lambda b, c: (b * LT + c, 0)),
            pl.BlockSpec((d_model, d_in_proj), lambda b, c: (0, 0)),
            pl.BlockSpec((d_conv, conv_dim), lambda b, c: (0, 0)),
            pl.BlockSpec((1, conv_dim), lambda b, c: (0, 0)),
            pl.BlockSpec((1, H), lambda b, c: (0, 0)),
            pl.BlockSpec((1, H), lambda b, c: (0, 0)),
            pl.BlockSpec((H, d_inner), lambda b, c: (0, 0)),
            pl.BlockSpec((1, d_inner), lambda b, c: (0, 0)),
            pl.BlockSpec((1, d_inner), lambda b, c: (0, 0)),
            pl.BlockSpec((d_inner, d_out), lambda b, c: (0, 0)),
        ],
        out_specs=pl.BlockSpec((Q, d_out), lambda b, c: (b * LT + c, 0)),
        out_shape=jax.ShapeDtypeStruct((M_total, d_out), jnp.float32),
        scratch_shapes=[
            pltpu.VMEM((8, conv_dim), jnp.float32),
            pltpu.VMEM((H * P // 256, N, 256), jnp.float32),
            pltpu.VMEM((Q, conv_dim), jnp.float32),
            pltpu.VMEM((Q, d_inner), jnp.float32),
            pltpu.VMEM((Q, d_inner), jnp.float32),
        ],
        compiler_params=pltpu.CompilerParams(
            dimension_semantics=("parallel", "arbitrary"),
            vmem_limit_bytes=56 * 1024 * 1024,
            fuse_transposed_lhs_in_matmul=True,
        ),
        name="mamba2_fused",
    )(xf, W_in_bf, cwT, cb2, dtb2, a_row2, e64, drep, nw2, W_out_bf)

    return out.reshape(Bsz, L, d_out)
